# baseline (speedup 1.0000x reference)
import jax
import jax.numpy as jnp
from jax import lax
from jax.experimental import pallas as pl
from jax.experimental.pallas import tpu as pltpu
from jax.experimental.pallas import tpu_sc as plsc

_W = 256
_NW = 32


def _lookup2(table, idx_e, idx_o):
    n_pairs = idx_e.shape[0]
    _, embed = table.shape
    per_sub = n_pairs // _NW
    n_win = per_sub // _W

    mesh = plsc.VectorSubcoreMesh(core_axis_name="core",
                                  subcore_axis_name="subcore")

    @pl.kernel(
        out_type=jax.ShapeDtypeStruct((n_pairs, 2 * embed), table.dtype),
        mesh=mesh,
        compiler_params=pltpu.CompilerParams(use_tc_tiling_on_sc=False),
        scratch_types=[
            pltpu.VMEM((per_sub,), jnp.int32),
            pltpu.VMEM((per_sub,), jnp.int32),
            pltpu.VMEM((2, _W, 64), jnp.float32),
            pltpu.VMEM((2, _W, 64), jnp.float32),
            pltpu.SemaphoreType.DMA,
            pltpu.SemaphoreType.DMA,
            pltpu.SemaphoreType.DMA,
        ],
    )
    def lookup(table_hbm, idxe_hbm, idxo_hbm, out_hbm,
               idxe_v, idxo_v, se, so, gsem, osem0, osem1):
        wid = lax.axis_index("subcore") * 2 + lax.axis_index("core")
        base = wid * per_sub

        pltpu.sync_copy(idxe_hbm.at[pl.ds(base, per_sub)], idxe_v)
        pltpu.sync_copy(idxo_hbm.at[pl.ds(base, per_sub)], idxo_v)

        def launch(w, slot):
            pltpu.async_copy(table_hbm.at[idxe_v.at[pl.ds(w * _W, _W)]],
                             se.at[slot], gsem)
            pltpu.async_copy(table_hbm.at[idxo_v.at[pl.ds(w * _W, _W)]],
                             so.at[slot], gsem)

        def out_refs(w):
            rows = pl.ds(base + w * _W, _W)
            return (out_hbm.at[rows, pl.ds(0, embed)],
                    out_hbm.at[rows, pl.ds(embed, embed)])

        launch(0, 0)

        def step(w, slot):
            pltpu.make_async_copy(table_hbm.at[idxe_v.at[pl.ds(0, _W)]],
                                  se.at[slot], gsem).wait()
            pltpu.make_async_copy(table_hbm.at[idxo_v.at[pl.ds(0, _W)]],
                                  so.at[slot], gsem).wait()

            @pl.when(w + 1 < n_win)
            def _():
                @pl.when(w >= 1)
                def _():
                    oe, oo = out_refs(w - 1)
                    other = osem1 if slot == 0 else osem0
                    pltpu.make_async_copy(se.at[1 - slot], oe, other).wait()
                    pltpu.make_async_copy(so.at[1 - slot], oo, other).wait()

                launch(w + 1, 1 - slot)

            oe, oo = out_refs(w)
            osem = osem0 if slot == 0 else osem1
            pltpu.async_copy(se.at[slot], oe, osem)
            pltpu.async_copy(so.at[slot], oo, osem)

        @pl.loop(0, n_win // 2)
        def _(h):
            step(2 * h, 0)
            step(2 * h + 1, 1)

        oe, oo = out_refs(n_win - 2)
        pltpu.make_async_copy(se.at[0], oe, osem0).wait()
        pltpu.make_async_copy(so.at[0], oo, osem0).wait()
        oe, oo = out_refs(n_win - 1)
        pltpu.make_async_copy(se.at[1], oe, osem1).wait()
        pltpu.make_async_copy(so.at[1], oo, osem1).wait()

    return lookup(table, idx_e, idx_o)


def kernel(input, table):
    batch, seq = input.shape
    _, embed = table.shape
    idx = input.astype(jnp.int32)
    out2 = _lookup2(table, idx[:, 0::2].reshape(-1), idx[:, 1::2].reshape(-1))
    return out2.reshape(batch, seq, embed)
